# baseline (speedup 1.0000x reference)
import functools

import jax
import jax.numpy as jnp
from jax import lax
from jax.experimental import pallas as pl
from jax.experimental.pallas import tpu as pltpu
from jax.experimental.pallas import tpu_sc as plsc

_N_CODES = 1024
_DIM = 32
_ROWS = 32768
_BLK = 1024
_NW = 32
_RPW = _ROWS // _NW
_GCH = 128


def _assign_body(x_ref, emb2_t_ref, codes_ref):
    x = x_ref[...]
    et2 = emb2_t_ref[...]
    x2 = jnp.sum(x * x, axis=1, keepdims=True)
    e2 = 0.25 * jnp.sum(et2 * et2, axis=0, keepdims=True)
    s2 = lax.dot_general(x, et2, (((1,), (0,)), ((), ())),
                         preferred_element_type=jnp.float32)
    d = (x2 - s2) + e2
    idx = jnp.argmin(d, axis=1).astype(jnp.int32)
    codes_ref[...] = idx.reshape(1, 1, _BLK)


def _assign(x, emb_t):
    return pl.pallas_call(
        _assign_body,
        grid=(_ROWS // _BLK,),
        in_specs=[
            pl.BlockSpec((_BLK, _DIM), lambda i: (i, 0)),
            pl.BlockSpec((_DIM, _N_CODES), lambda i: (0, 0)),
        ],
        out_specs=pl.BlockSpec((1, 1, _BLK), lambda i: (i, 0, 0)),
        out_shape=jax.ShapeDtypeStruct((_ROWS // _BLK, 1, _BLK), jnp.int32),
    )(x, emb_t)


def _sc_body(x_hbm, emb_hbm, codes_hbm, q_hbm, part_hbm,
             idx_v, rows_v, x_v, acc_v, sem):
    wid = lax.axis_index("s") * 2 + lax.axis_index("c")
    base = wid * _RPW
    pltpu.sync_copy(codes_hbm.at[pl.ds(base, _RPW)], idx_v)
    copies = []
    for j in range(_RPW // _GCH):
        copies.append(pltpu.async_copy(
            emb_hbm.at[idx_v.at[pl.ds(j * _GCH, _GCH)]],
            rows_v.at[pl.ds(j * _GCH, _GCH)], sem))
    pltpu.sync_copy(x_hbm.at[pl.ds(base, _RPW)], x_v)
    for c in copies:
        c.wait()

    def row_step(r, acc):
        q0 = rows_v[r, pl.ds(0, 16)]
        q1 = rows_v[r, pl.ds(16, 16)]
        x0 = x_v[r, pl.ds(0, 16)]
        x1 = x_v[r, pl.ds(16, 16)]
        d0 = q0 - x0
        d1 = q1 - x1
        rows_v[r, pl.ds(0, 16)] = x0 + d0
        rows_v[r, pl.ds(16, 16)] = x1 + d1
        return acc + d0 * d0 + d1 * d1

    acc = lax.fori_loop(0, _RPW, row_step, jnp.zeros((16,), jnp.float32))
    acc_v[...] = acc
    pltpu.sync_copy(rows_v, q_hbm.at[pl.ds(base, _RPW)])
    pltpu.sync_copy(acc_v, part_hbm.at[wid])


def _sc_body_d2(emb_hbm, codes_hbm, q_hbm, idx_v, rows_v, sem):
    wid = lax.axis_index("s") * 2 + lax.axis_index("c")
    base = wid * _RPW
    pltpu.sync_copy(codes_hbm.at[pl.ds(base, _RPW)], idx_v)
    copies = []
    for j in range(_RPW // _GCH):
        copies.append(pltpu.async_copy(
            emb_hbm.at[idx_v.at[pl.ds(j * _GCH, _GCH)]],
            rows_v.at[pl.ds(j * _GCH, _GCH)], sem))
    for c in copies:
        c.wait()
    pltpu.sync_copy(rows_v, q_hbm.at[pl.ds(base, _RPW)])


@functools.cache
def _sc_gather_d2():
    return pl.kernel(
        _sc_body_d2,
        out_type=jax.ShapeDtypeStruct((_ROWS, _DIM), jnp.float32),
        mesh=plsc.VectorSubcoreMesh(core_axis_name="c", subcore_axis_name="s"),
        compiler_params=pltpu.CompilerParams(use_tc_tiling_on_sc=False),
        scratch_types=[
            pltpu.VMEM((_RPW,), jnp.int32),
            pltpu.VMEM((_RPW, _DIM), jnp.float32),
            pltpu.SemaphoreType.DMA,
        ],
    )


@functools.cache
def _sc_gather():
    return pl.kernel(
        _sc_body,
        out_type=(
            jax.ShapeDtypeStruct((_ROWS, _DIM), jnp.float32),
            jax.ShapeDtypeStruct((_NW, 16), jnp.float32),
        ),
        mesh=plsc.VectorSubcoreMesh(core_axis_name="c", subcore_axis_name="s"),
        compiler_params=pltpu.CompilerParams(use_tc_tiling_on_sc=False),
        scratch_types=[
            pltpu.VMEM((_RPW,), jnp.int32),
            pltpu.VMEM((_RPW, _DIM), jnp.float32),
            pltpu.VMEM((_RPW, _DIM), jnp.float32),
            pltpu.VMEM((16,), jnp.float32),
            pltpu.SemaphoreType.DMA,
        ],
    )


@jax.jit
def kernel(x, embedding):
    codes3 = _assign(x, (embedding * 2.0).T)
    codes = codes3.reshape(_ROWS)
    q = _sc_gather_d2()(embedding, codes)
    return q, jnp.float32(0), codes.reshape(_ROWS, 1)


# speedup vs baseline: 1.3404x; 1.3404x over previous
import functools

import jax
import jax.numpy as jnp
from jax import lax
from jax.experimental import pallas as pl
from jax.experimental.pallas import tpu as pltpu
from jax.experimental.pallas import tpu_sc as plsc

_N_CODES = 1024
_DIM = 32
_ROWS = 32768
_BLK = 1024
_NW = 32
_RPW = _ROWS // _NW
_GCH = 128


def _assign_body(xt_ref, emb2_ref, codes_ref, dsum_ref):
    i = pl.program_id(0)
    xb = xt_ref[...]
    eb = emb2_ref[...]
    x2 = jnp.sum(xb * xb, axis=0, keepdims=True)
    e2 = 0.25 * jnp.sum(eb * eb, axis=1, keepdims=True)
    s2 = lax.dot_general(eb, xb, (((1,), (0,)), ((), ())),
                         preferred_element_type=jnp.float32)
    dT = (x2 - s2) + e2
    idx = jnp.argmin(dT, axis=0).astype(jnp.int32)
    codes_ref[...] = idx.reshape(1, 1, _BLK)
    part = jnp.sum(jnp.min(dT, axis=0)).reshape(1, 1)

    @pl.when(i == 0)
    def _init():
        dsum_ref[...] = part

    @pl.when(i != 0)
    def _acc():
        dsum_ref[...] = dsum_ref[...] + part


def _assign(xt, emb2):
    return pl.pallas_call(
        _assign_body,
        grid=(_ROWS // _BLK,),
        in_specs=[
            pl.BlockSpec((_DIM, _BLK), lambda i: (0, i)),
            pl.BlockSpec((_N_CODES, _DIM), lambda i: (0, 0)),
        ],
        out_specs=[
            pl.BlockSpec((1, 1, _BLK), lambda i: (i, 0, 0)),
            pl.BlockSpec((1, 1), lambda i: (0, 0)),
        ],
        out_shape=[
            jax.ShapeDtypeStruct((_ROWS // _BLK, 1, _BLK), jnp.int32),
            jax.ShapeDtypeStruct((1, 1), jnp.float32),
        ],
    )(xt, emb2)


def _sc_body(emb_hbm, codes_hbm, q_hbm, idx_v, rows_v, sem):
    wid = lax.axis_index("s") * 2 + lax.axis_index("c")
    base = wid * _RPW
    pltpu.sync_copy(codes_hbm.at[pl.ds(base, _RPW)], idx_v)
    copies = []
    for j in range(_RPW // _GCH):
        copies.append(pltpu.async_copy(
            emb_hbm.at[idx_v.at[pl.ds(j * _GCH, _GCH)]],
            rows_v.at[pl.ds(j * _GCH, _GCH)], sem))
    for c in copies:
        c.wait()
    pltpu.sync_copy(rows_v, q_hbm.at[pl.ds(base, _RPW)])


@functools.cache
def _sc_gather():
    return pl.kernel(
        _sc_body,
        out_type=jax.ShapeDtypeStruct((_ROWS, _DIM), jnp.float32),
        mesh=plsc.VectorSubcoreMesh(core_axis_name="c", subcore_axis_name="s"),
        compiler_params=pltpu.CompilerParams(use_tc_tiling_on_sc=False),
        scratch_types=[
            pltpu.VMEM((_RPW,), jnp.int32),
            pltpu.VMEM((_RPW, _DIM), jnp.float32),
            pltpu.SemaphoreType.DMA,
        ],
    )


@jax.jit
def kernel(x, embedding):
    xt = x.T
    codes3, dsum = _assign(xt, embedding * 2.0)
    codes = codes3.reshape(_ROWS)
    quantize_st = _sc_gather()(embedding, codes)
    diff = dsum[0, 0] / jnp.float32(_ROWS * _DIM)
    return quantize_st, diff, codes.reshape(_ROWS, 1)


# speedup vs baseline: 1.3871x; 1.0348x over previous
import functools

import jax
import jax.numpy as jnp
from jax import lax
from jax.experimental import pallas as pl
from jax.experimental.pallas import tpu as pltpu
from jax.experimental.pallas import tpu_sc as plsc

_N_CODES = 1024
_DIM = 32
_ROWS = 32768
_BLK = 1024
_NCH = 2
_CROWS = _ROWS // _NCH
_NW = 32
_RPW = _CROWS // _NW
_GCH = 128


def _assign_body(xt_ref, emb2_ref, codes_ref, dsum_ref):
    i = pl.program_id(0)
    xb = xt_ref[...]
    eb = emb2_ref[...]
    x2 = jnp.sum(xb * xb, axis=0, keepdims=True)
    e2 = 0.25 * jnp.sum(eb * eb, axis=1, keepdims=True)
    s2 = lax.dot_general(eb, xb, (((1,), (0,)), ((), ())),
                         preferred_element_type=jnp.float32)
    dT = (x2 - s2) + e2
    idx = jnp.argmin(dT, axis=0).astype(jnp.int32)
    codes_ref[...] = idx.reshape(1, 1, _BLK)
    part = jnp.sum(jnp.min(dT, axis=0)).reshape(1, 1)

    @pl.when(i == 0)
    def _init():
        dsum_ref[...] = part

    @pl.when(i != 0)
    def _acc():
        dsum_ref[...] = dsum_ref[...] + part


def _assign(xt, emb2, chunk):
    base = chunk * (_CROWS // _BLK)
    return pl.pallas_call(
        _assign_body,
        grid=(_CROWS // _BLK,),
        in_specs=[
            pl.BlockSpec((_DIM, _BLK), lambda i: (0, base + i)),
            pl.BlockSpec((_N_CODES, _DIM), lambda i: (0, 0)),
        ],
        out_specs=[
            pl.BlockSpec((1, 1, _BLK), lambda i: (i, 0, 0)),
            pl.BlockSpec((1, 1), lambda i: (0, 0)),
        ],
        out_shape=[
            jax.ShapeDtypeStruct((_CROWS // _BLK, 1, _BLK), jnp.int32),
            jax.ShapeDtypeStruct((1, 1), jnp.float32),
        ],
    )(xt, emb2)


def _sc_body(emb_hbm, codes_hbm, q_hbm, idx_v, rows_v, sem):
    wid = lax.axis_index("s") * 2 + lax.axis_index("c")
    base = wid * _RPW
    pltpu.sync_copy(codes_hbm.at[pl.ds(base, _RPW)], idx_v)
    copies = []
    for j in range(_RPW // _GCH):
        copies.append(pltpu.async_copy(
            emb_hbm.at[idx_v.at[pl.ds(j * _GCH, _GCH)]],
            rows_v.at[pl.ds(j * _GCH, _GCH)], sem))
    for c in copies:
        c.wait()
    pltpu.sync_copy(rows_v, q_hbm.at[pl.ds(base, _RPW)])


@functools.cache
def _sc_gather():
    return pl.kernel(
        _sc_body,
        out_type=jax.ShapeDtypeStruct((_CROWS, _DIM), jnp.float32),
        mesh=plsc.VectorSubcoreMesh(core_axis_name="c", subcore_axis_name="s"),
        compiler_params=pltpu.CompilerParams(use_tc_tiling_on_sc=False),
        scratch_types=[
            pltpu.VMEM((_RPW,), jnp.int32),
            pltpu.VMEM((_RPW, _DIM), jnp.float32),
            pltpu.SemaphoreType.DMA,
        ],
    )


@jax.jit
def kernel(x, embedding):
    xt = x.T
    emb2 = embedding * 2.0
    codes_parts = []
    dsum_parts = []
    q_parts = []
    for c in range(_NCH):
        codes3, dsum = _assign(xt, emb2, c)
        codes_parts.append(codes3.reshape(_CROWS))
        dsum_parts.append(dsum[0, 0])
        q_parts.append(_sc_gather()(embedding, codes_parts[-1]))
    quantize_st = jnp.concatenate(q_parts, axis=0)
    codes = jnp.concatenate(codes_parts, axis=0)
    diff = sum(dsum_parts) / jnp.float32(_ROWS * _DIM)
    return quantize_st, diff, codes.reshape(_ROWS, 1)


# speedup vs baseline: 1.3928x; 1.0041x over previous
import functools

import jax
import jax.numpy as jnp
from jax import lax
from jax.experimental import pallas as pl
from jax.experimental.pallas import tpu as pltpu
from jax.experimental.pallas import tpu_sc as plsc

_N_CODES = 1024
_DIM = 32
_ROWS = 32768
_BLK = 1024
_NCH = 4
_CROWS = _ROWS // _NCH
_NW = 32
_RPW = _CROWS // _NW
_GCH = 128


def _assign_body(xt_ref, emb2_ref, codes_ref, dsum_ref):
    i = pl.program_id(0)
    xb = xt_ref[...]
    eb = emb2_ref[...]
    x2 = jnp.sum(xb * xb, axis=0, keepdims=True)
    e2 = 0.25 * jnp.sum(eb * eb, axis=1, keepdims=True)
    s2 = lax.dot_general(eb, xb, (((1,), (0,)), ((), ())),
                         preferred_element_type=jnp.float32)
    dT = (x2 - s2) + e2
    idx = jnp.argmin(dT, axis=0).astype(jnp.int32)
    codes_ref[...] = idx.reshape(1, 1, _BLK)
    part = jnp.sum(jnp.min(dT, axis=0)).reshape(1, 1)

    @pl.when(i == 0)
    def _init():
        dsum_ref[...] = part

    @pl.when(i != 0)
    def _acc():
        dsum_ref[...] = dsum_ref[...] + part


def _assign(xt, emb2, chunk):
    base = chunk * (_CROWS // _BLK)
    return pl.pallas_call(
        _assign_body,
        grid=(_CROWS // _BLK,),
        in_specs=[
            pl.BlockSpec((_DIM, _BLK), lambda i: (0, base + i)),
            pl.BlockSpec((_N_CODES, _DIM), lambda i: (0, 0)),
        ],
        out_specs=[
            pl.BlockSpec((1, 1, _BLK), lambda i: (i, 0, 0)),
            pl.BlockSpec((1, 1), lambda i: (0, 0)),
        ],
        out_shape=[
            jax.ShapeDtypeStruct((_CROWS // _BLK, 1, _BLK), jnp.int32),
            jax.ShapeDtypeStruct((1, 1), jnp.float32),
        ],
    )(xt, emb2)


def _sc_body(emb_hbm, codes_hbm, q_hbm, idx_v, rows_v, sem):
    wid = lax.axis_index("s") * 2 + lax.axis_index("c")
    base = wid * _RPW
    pltpu.sync_copy(codes_hbm.at[pl.ds(base, _RPW)], idx_v)
    copies = []
    for j in range(_RPW // _GCH):
        copies.append(pltpu.async_copy(
            emb_hbm.at[idx_v.at[pl.ds(j * _GCH, _GCH)]],
            rows_v.at[pl.ds(j * _GCH, _GCH)], sem))
    for c in copies:
        c.wait()
    pltpu.sync_copy(rows_v, q_hbm.at[pl.ds(base, _RPW)])


@functools.cache
def _sc_gather():
    return pl.kernel(
        _sc_body,
        out_type=jax.ShapeDtypeStruct((_CROWS, _DIM), jnp.float32),
        mesh=plsc.VectorSubcoreMesh(core_axis_name="c", subcore_axis_name="s"),
        compiler_params=pltpu.CompilerParams(use_tc_tiling_on_sc=False),
        scratch_types=[
            pltpu.VMEM((_RPW,), jnp.int32),
            pltpu.VMEM((_RPW, _DIM), jnp.float32),
            pltpu.SemaphoreType.DMA,
        ],
    )


@jax.jit
def kernel(x, embedding):
    xt = x.T
    emb2 = embedding * 2.0
    codes_parts = []
    dsum_parts = []
    q_parts = []
    for c in range(_NCH):
        codes3, dsum = _assign(xt, emb2, c)
        codes_parts.append(codes3.reshape(_CROWS))
        dsum_parts.append(dsum[0, 0])
        q_parts.append(_sc_gather()(embedding, codes_parts[-1]))
    quantize_st = jnp.concatenate(q_parts, axis=0)
    codes = jnp.concatenate(codes_parts, axis=0)
    diff = sum(dsum_parts) / jnp.float32(_ROWS * _DIM)
    return quantize_st, diff, codes.reshape(_ROWS, 1)


# speedup vs baseline: 1.4835x; 1.0652x over previous
import functools

import jax
import jax.numpy as jnp
from jax import lax
from jax.experimental import pallas as pl
from jax.experimental.pallas import tpu as pltpu
from jax.experimental.pallas import tpu_sc as plsc

_N_CODES = 1024
_DIM = 32
_ROWS = 32768
_BLK = 4096
_NCH = 2
_CROWS = _ROWS // _NCH
_NW = 32
_RPW = _CROWS // _NW
_GCH = 128


def _assign_body(xt_ref, emb2_ref, codes_ref, dsum_ref):
    i = pl.program_id(0)
    xb = xt_ref[...]
    eb = emb2_ref[...]
    x2 = jnp.sum(xb * xb, axis=0, keepdims=True)
    e2 = 0.25 * jnp.sum(eb * eb, axis=1, keepdims=True)
    s2 = lax.dot_general(eb, xb, (((1,), (0,)), ((), ())),
                         preferred_element_type=jnp.float32)
    dT = (x2 - s2) + e2
    idx = jnp.argmin(dT, axis=0).astype(jnp.int32)
    codes_ref[...] = idx.reshape(1, 1, _BLK)
    part = jnp.sum(jnp.min(dT, axis=0)).reshape(1, 1)

    @pl.when(i == 0)
    def _init():
        dsum_ref[...] = part

    @pl.when(i != 0)
    def _acc():
        dsum_ref[...] = dsum_ref[...] + part


def _assign(xt, emb2, chunk):
    base = chunk * (_CROWS // _BLK)
    return pl.pallas_call(
        _assign_body,
        grid=(_CROWS // _BLK,),
        in_specs=[
            pl.BlockSpec((_DIM, _BLK), lambda i: (0, base + i)),
            pl.BlockSpec((_N_CODES, _DIM), lambda i: (0, 0)),
        ],
        out_specs=[
            pl.BlockSpec((1, 1, _BLK), lambda i: (i, 0, 0)),
            pl.BlockSpec((1, 1), lambda i: (0, 0)),
        ],
        out_shape=[
            jax.ShapeDtypeStruct((_CROWS // _BLK, 1, _BLK), jnp.int32),
            jax.ShapeDtypeStruct((1, 1), jnp.float32),
        ],
    )(xt, emb2)


def _sc_body(emb_hbm, codes_hbm, q_hbm, idx_v, rows_v, sem):
    wid = lax.axis_index("s") * 2 + lax.axis_index("c")
    base = wid * _RPW
    pltpu.sync_copy(codes_hbm.at[pl.ds(base, _RPW)], idx_v)
    copies = []
    for j in range(_RPW // _GCH):
        copies.append(pltpu.async_copy(
            emb_hbm.at[idx_v.at[pl.ds(j * _GCH, _GCH)]],
            rows_v.at[pl.ds(j * _GCH, _GCH)], sem))
    for c in copies:
        c.wait()
    pltpu.sync_copy(rows_v, q_hbm.at[pl.ds(base, _RPW)])


@functools.cache
def _sc_gather():
    return pl.kernel(
        _sc_body,
        out_type=jax.ShapeDtypeStruct((_CROWS, _DIM), jnp.float32),
        mesh=plsc.VectorSubcoreMesh(core_axis_name="c", subcore_axis_name="s"),
        compiler_params=pltpu.CompilerParams(use_tc_tiling_on_sc=False),
        scratch_types=[
            pltpu.VMEM((_RPW,), jnp.int32),
            pltpu.VMEM((_RPW, _DIM), jnp.float32),
            pltpu.SemaphoreType.DMA,
        ],
    )


@jax.jit
def kernel(x, embedding):
    xt = x.T
    emb2 = embedding * 2.0
    codes_parts = []
    dsum_parts = []
    q_parts = []
    for c in range(_NCH):
        codes3, dsum = _assign(xt, emb2, c)
        codes_parts.append(codes3.reshape(_CROWS))
        dsum_parts.append(dsum[0, 0])
        q_parts.append(_sc_gather()(embedding, codes_parts[-1]))
    quantize_st = jnp.concatenate(q_parts, axis=0)
    codes = jnp.concatenate(codes_parts, axis=0)
    diff = sum(dsum_parts) / jnp.float32(_ROWS * _DIM)
    return quantize_st, diff, codes.reshape(_ROWS, 1)
